# baseline (speedup 1.0000x reference)
import functools

import jax
import jax.numpy as jnp
from jax import lax
from jax.experimental import pallas as pl
from jax.experimental.pallas import tpu as pltpu
from jax.experimental.pallas import tpu_sc as plsc

VOCAB = 100000
FROM_DIM = 128
TO_DIM = 128
BATCH = 4096
HIST = 50

_ROWS_BLK = 25000


def _proj_body(t_ref, w_ref, b_ref, o_ref):
    o_ref[...] = lax.dot_general(
        t_ref[...], w_ref[...],
        dimension_numbers=(((1,), (1,)), ((), ())),
        preferred_element_type=jnp.float32,
    ) + b_ref[...]


_project = pl.pallas_call(
    _proj_body,
    grid=(VOCAB // _ROWS_BLK,),
    in_specs=[
        pl.BlockSpec((_ROWS_BLK, FROM_DIM), lambda i: (i, 0)),
        pl.BlockSpec((TO_DIM, FROM_DIM), lambda i: (0, 0)),
        pl.BlockSpec((1, TO_DIM), lambda i: (0, 0)),
    ],
    out_specs=pl.BlockSpec((_ROWS_BLK, TO_DIM), lambda i: (i, 0)),
    out_shape=jax.ShapeDtypeStruct((VOCAB, TO_DIM), jnp.float32),
)

_NC, _NS = 2, 16
_NW = _NC * _NS
_CH = 128
_B_TOT = BATCH * HIST
_CPW = _B_TOT // _CH // _NW
_NBUF = 5
_LEAD = 3
_WSLACK = _NBUF - _LEAD


def _gather_body(p_hbm, idx_hbm, out_hbm, idx_v, buf, gsem, wsem):
    wid = lax.axis_index("s") * _NC + lax.axis_index("c")
    row0 = wid * _CPW * _CH

    pltpu.sync_copy(idx_hbm.at[wid], idx_v)

    def start_gather(j, slot):
        pltpu.async_copy(p_hbm.at[idx_v.at[j]], buf.at[slot], gsem.at[slot])

    def wait_gather(j, slot):
        pltpu.make_async_copy(
            p_hbm.at[idx_v.at[j]], buf.at[slot], gsem.at[slot]).wait()

    def start_write(j, slot):
        pltpu.async_copy(
            buf.at[slot], out_hbm.at[pl.ds(row0 + j * _CH, _CH)],
            wsem.at[slot])

    def wait_write(j, slot):
        pltpu.make_async_copy(
            buf.at[slot], out_hbm.at[pl.ds(row0 + j * _CH, _CH)],
            wsem.at[slot]).wait()

    for s in range(_LEAD):
        start_gather(s, s)

    def outer(i, carry):
        for s in range(_NBUF):
            j = i * _NBUF + s
            jl = j + _LEAD
            slot_l = (s + _LEAD) % _NBUF

            @pl.when(jnp.logical_and(j >= _WSLACK, jl < _CPW))
            def _():
                wait_write(j - _WSLACK, slot_l)

            @pl.when(jl < _CPW)
            def _():
                start_gather(jl, slot_l)

            wait_gather(j, s)
            start_write(j, s)
        return carry

    lax.fori_loop(0, _CPW // _NBUF, outer, 0)

    for s in range(_NBUF):
        wait_write(_CPW - _NBUF + s, s)


_gather = functools.partial(
    pl.kernel,
    out_type=jax.ShapeDtypeStruct((_B_TOT, TO_DIM), jnp.float32),
    mesh=plsc.VectorSubcoreMesh(core_axis_name="c", subcore_axis_name="s",
                                num_cores=_NC, num_subcores=_NS),
    scratch_types=[
        pltpu.VMEM((_CPW, _CH), jnp.int32),
        pltpu.VMEM((_NBUF, _CH, TO_DIM), jnp.float32),
        pltpu.SemaphoreType.DMA((_NBUF,)),
        pltpu.SemaphoreType.DMA((_NBUF,)),
    ],
)(_gather_body)


def kernel(indexes, table, W, b):
    P = _project(table, W, b.reshape(1, TO_DIM))
    idx = jnp.transpose(indexes).astype(jnp.int32).reshape(_NW, _CPW, _CH)
    out = _gather(P, idx)
    return out.reshape(HIST, BATCH, TO_DIM).transpose(1, 0, 2)


# speedup vs baseline: 1.0230x; 1.0230x over previous
import functools

import jax
import jax.numpy as jnp
from jax import lax
from jax.experimental import pallas as pl
from jax.experimental.pallas import tpu as pltpu
from jax.experimental.pallas import tpu_sc as plsc

VOCAB = 100000
FROM_DIM = 128
TO_DIM = 128
BATCH = 4096
HIST = 50

_ROWS_BLK = 20000


def _proj_body(t_ref, w_ref, b_ref, o_ref):
    o_ref[...] = lax.dot_general(
        t_ref[...], w_ref[...],
        dimension_numbers=(((1,), (1,)), ((), ())),
        preferred_element_type=jnp.float32,
    ) + b_ref[...]


_project = pl.pallas_call(
    _proj_body,
    grid=(VOCAB // _ROWS_BLK,),
    in_specs=[
        pl.BlockSpec((_ROWS_BLK, FROM_DIM), lambda i: (i, 0)),
        pl.BlockSpec((TO_DIM, FROM_DIM), lambda i: (0, 0)),
        pl.BlockSpec((1, TO_DIM), lambda i: (0, 0)),
    ],
    out_specs=pl.BlockSpec((_ROWS_BLK, TO_DIM), lambda i: (i, 0)),
    out_shape=jax.ShapeDtypeStruct((VOCAB, TO_DIM), jnp.float32),
)

_NC, _NS = 2, 16
_NW = _NC * _NS
_CH = 128
_B_TOT = BATCH * HIST
_CPW = _B_TOT // _CH // _NW
_NBUF = 5
_LEAD = 3
_WSLACK = _NBUF - _LEAD


def _gather_body(p_hbm, idx_hbm, out_hbm, idx_v, buf, gsem, wsem):
    wid = lax.axis_index("s") * _NC + lax.axis_index("c")
    row0 = wid * _CPW * _CH

    pltpu.sync_copy(idx_hbm.at[pl.ds(row0, _CPW * _CH)], idx_v)

    def start_gather(j, slot):
        pltpu.async_copy(p_hbm.at[idx_v.at[pl.ds(j * _CH, _CH)]],
                         buf.at[slot], gsem.at[slot])

    def wait_gather(j, slot):
        pltpu.make_async_copy(
            p_hbm.at[idx_v.at[pl.ds(j * _CH, _CH)]],
            buf.at[slot], gsem.at[slot]).wait()

    def start_write(j, slot):
        pltpu.async_copy(
            buf.at[slot], out_hbm.at[pl.ds(row0 + j * _CH, _CH)],
            wsem.at[slot])

    def wait_write(j, slot):
        pltpu.make_async_copy(
            buf.at[slot], out_hbm.at[pl.ds(row0 + j * _CH, _CH)],
            wsem.at[slot]).wait()

    for s in range(_LEAD):
        start_gather(s, s)

    def outer(i, carry):
        for s in range(_NBUF):
            j = i * _NBUF + s
            jl = j + _LEAD
            slot_l = (s + _LEAD) % _NBUF

            @pl.when(jnp.logical_and(j >= _WSLACK, jl < _CPW))
            def _():
                wait_write(j - _WSLACK, slot_l)

            @pl.when(jl < _CPW)
            def _():
                start_gather(jl, slot_l)

            wait_gather(j, s)
            start_write(j, s)
        return carry

    lax.fori_loop(0, _CPW // _NBUF, outer, 0)

    for s in range(_NBUF):
        wait_write(_CPW - _NBUF + s, s)


_gather = functools.partial(
    pl.kernel,
    out_type=jax.ShapeDtypeStruct((_B_TOT, TO_DIM), jnp.float32),
    mesh=plsc.VectorSubcoreMesh(core_axis_name="c", subcore_axis_name="s",
                                num_cores=_NC, num_subcores=_NS),
    scratch_types=[
        pltpu.VMEM((_CPW * _CH,), jnp.int32),
        pltpu.VMEM((_NBUF, _CH, TO_DIM), jnp.float32),
        pltpu.SemaphoreType.DMA((_NBUF,)),
        pltpu.SemaphoreType.DMA((_NBUF,)),
    ],
)(_gather_body)


def kernel(indexes, table, W, b):
    P = _project(table, W, b.reshape(1, TO_DIM))
    idx = jnp.transpose(indexes).astype(jnp.int32).reshape(_B_TOT)
    out = _gather(P, idx)
    return out.reshape(HIST, BATCH, TO_DIM).transpose(1, 0, 2)
